# Initial kernel scaffold; baseline (speedup 1.0000x reference)
import jax
import jax.numpy as jnp
from jax.experimental import pallas as pl


def kernel(emb, WxT_ctx, WxT_emb, bx, Wih4T, Whh4T, b_lstm4, WsT_h, WsT_c, bs, v_att, Wp_ctx, Wp_h, Wp_c, Wp_x, bp, W1T_h, W1T_ctx, b1, W2T, b2, enc, encf, mask, decoder_input, h0, c0, previous_context_vector, extra_zeros, encoder_input_extra_vocabs):
    raise NotImplementedError("write your pallas kernel here")


# speedup vs baseline: 1.0034x; 1.0034x over previous
import jax
import jax.numpy as jnp
from jax.experimental import pallas as pl
from jax.experimental.pallas import tpu as pltpu


def _state_kernel(emb_ref, ctxprev_ref, h0_ref, c0_ref,
                  enc_ref, encf_ref, mask_ref,
                  wxt_ctx_ref, wxt_emb_ref, bx_ref,
                  wih4_ref, whh4_ref, b4_ref,
                  wst_h_ref, wst_c_ref, bs_ref, v_ref,
                  wp_ctx_ref, wp_h_ref, wp_c_ref, wp_x_ref, bp_ref,
                  w1t_h_ref, w1t_ctx_ref, b1_ref,
                  h_ref, c_ref, ctx_ref, attn_ref, attn_oov_ref,
                  pgen_ref, hidden_ref,
                  x_sc, decf_sc, m_sc, l_sc, ctxacc_sc, scores_sc):
    f32, bf16 = jnp.float32, jnp.bfloat16
    t = pl.program_id(1)
    n_t = pl.num_programs(1)
    TT = enc_ref.shape[1]

    def mm(a, w_ref):
        return jnp.dot(a.astype(bf16), w_ref[...], preferred_element_type=f32)

    @pl.when(t == 0)
    def _():
        x = (mm(ctxprev_ref[...], wxt_ctx_ref) + mm(emb_ref[...], wxt_emb_ref)
             + bx_ref[...])

        def gate(g):
            return (jnp.dot(x.astype(bf16), wih4_ref[g], preferred_element_type=f32)
                    + jnp.dot(h0_ref[...].astype(bf16), whh4_ref[g],
                              preferred_element_type=f32)
                    + b4_ref[g])

        i_g = jax.nn.sigmoid(gate(0))
        f_g = jax.nn.sigmoid(gate(1))
        g_g = jnp.tanh(gate(2))
        o_g = jax.nn.sigmoid(gate(3))
        c = f_g * c0_ref[...] + i_g * g_g
        h = o_g * jnp.tanh(c)
        h_ref[...] = h
        c_ref[...] = c
        x_sc[...] = x
        decf_sc[...] = mm(h, wst_h_ref) + mm(c, wst_c_ref) + bs_ref[...]
        m_sc[...] = jnp.full(m_sc.shape, -1e30, f32)
        l_sc[...] = jnp.zeros(l_sc.shape, f32)
        ctxacc_sc[...] = jnp.zeros(ctxacc_sc.shape, f32)

    start = pl.multiple_of(t * TT, TT)
    mask_t = mask_ref[:, pl.ds(start, TT)]
    energy = jnp.tanh(encf_ref[...].astype(f32) + decf_sc[...][:, None, :])
    scores = jnp.sum(energy * v_ref[...][None, :, :], axis=-1)
    scores = jnp.where(mask_t > 0.0, scores, -1e30)
    scores_sc[:, pl.ds(start, TT)] = scores

    m_old = m_sc[...]
    m_new = jnp.maximum(m_old, jnp.max(scores, axis=-1, keepdims=True))
    corr = jnp.exp(m_old - m_new)
    p = jnp.exp(scores - m_new)
    l_sc[...] = l_sc[...] * corr + jnp.sum(p, axis=-1, keepdims=True)
    ctxacc_sc[...] = (ctxacc_sc[...] * corr
                      + jnp.sum(p[:, :, None] * enc_ref[...].astype(f32), axis=1))
    m_sc[...] = m_new

    @pl.when(t == n_t - 1)
    def _():
        inv_l = 1.0 / jnp.maximum(l_sc[...], 1e-9)
        ctx = ctxacc_sc[...] * inv_l
        attn = jnp.exp(scores_sc[...] - m_sc[...]) * inv_l
        h = h_ref[...]
        c = c_ref[...]
        x = x_sc[...]
        p_lin = (jnp.sum(ctx * wp_ctx_ref[...], axis=-1, keepdims=True)
                 + jnp.sum(h * wp_h_ref[...], axis=-1, keepdims=True)
                 + jnp.sum(c * wp_c_ref[...], axis=-1, keepdims=True)
                 + jnp.sum(x * wp_x_ref[...], axis=-1, keepdims=True)
                 + bp_ref[...])
        p_gen = jax.nn.sigmoid(p_lin)
        ctx_ref[...] = ctx
        attn_ref[...] = attn
        attn_oov_ref[...] = (1.0 - p_gen) * attn
        pgen_ref[...] = p_gen
        hidden_ref[...] = mm(h, w1t_h_ref) + mm(ctx, w1t_ctx_ref) + b1_ref[...]


def _make_vocab_kernel(vocab_size):
    def _vocab_kernel(hidden_ref, pgen_ref, w2t_ref, b2_ref, out_ref, m_sc):
        f32, bf16 = jnp.float32, jnp.bfloat16
        j = pl.program_id(1)
        nv = pl.num_programs(1)
        TV = w2t_ref.shape[1]

        logits = (jnp.dot(hidden_ref[...].astype(bf16), w2t_ref[...],
                          preferred_element_type=f32) + b2_ref[...])

        @pl.when(j == 0)
        def _():
            m_sc[...] = jnp.full(m_sc.shape, -1e30, f32)

        m_sc[...] = jnp.maximum(m_sc[...], jnp.max(logits, axis=-1, keepdims=True))
        out_ref[:, pl.ds(pl.multiple_of(j * TV, TV), TV)] = logits

        @pl.when(j == nv - 1)
        def _():
            all_logits = out_ref[...]
            col = jax.lax.broadcasted_iota(jnp.int32, all_logits.shape, 1)
            probs = jnp.where(col < vocab_size,
                              jnp.exp(all_logits - m_sc[...]), 0.0)
            l = jnp.maximum(jnp.sum(probs, axis=-1, keepdims=True), 1e-30)
            out_ref[...] = probs * (pgen_ref[...] / l)

    return _vocab_kernel


def kernel(emb, WxT_ctx, WxT_emb, bx, Wih4T, Whh4T, b_lstm4, WsT_h, WsT_c, bs,
           v_att, Wp_ctx, Wp_h, Wp_c, Wp_x, bp, W1T_h, W1T_ctx, b1, W2T, b2,
           enc, encf, mask, decoder_input, h0, c0, previous_context_vector,
           extra_zeros, encoder_input_extra_vocabs):
    f32 = jnp.float32
    V, E = emb.shape
    B, Tpad, twoD = enc.shape
    D = twoD // 2
    T = encoder_input_extra_vocabs.shape[1]
    n_extra = extra_zeros.shape[1]
    Wout = b2.shape[1]
    TV = 4096
    nv = Wout // TV
    TT = 128
    nT = Tpad // TT
    BH = B // 2

    embedded = emb[decoder_input].astype(f32)
    ctxp = previous_context_vector.astype(f32)
    h0_2 = h0[0].astype(f32)
    c0_2 = c0[0].astype(f32)

    def half_spec(a):
        nd = a.ndim
        return pl.BlockSpec((BH,) + a.shape[1:],
                            lambda core, t, _nd=nd: (core,) + (0,) * (_nd - 1))

    def const_spec(a):
        nd = a.ndim
        return pl.BlockSpec(a.shape, lambda core, t, _nd=nd: (0,) * _nd)

    state_inputs = (
        embedded, ctxp, h0_2, c0_2,
        enc, encf, mask,
        WxT_ctx, WxT_emb, bx,
        Wih4T, Whh4T, b_lstm4,
        WsT_h, WsT_c, bs, v_att,
        Wp_ctx, Wp_h, Wp_c, Wp_x, bp,
        W1T_h, W1T_ctx, b1,
    )
    in_specs = ([half_spec(a) for a in state_inputs[:4]]
                + [pl.BlockSpec((BH, TT, twoD), lambda core, t: (core, t, 0)),
                   pl.BlockSpec((BH, TT, twoD), lambda core, t: (core, t, 0)),
                   half_spec(mask)]
                + [const_spec(a) for a in state_inputs[7:]])

    state_out_shapes = (
        jax.ShapeDtypeStruct((B, D), f32),
        jax.ShapeDtypeStruct((B, D), f32),
        jax.ShapeDtypeStruct((B, twoD), f32),
        jax.ShapeDtypeStruct((B, Tpad), f32),
        jax.ShapeDtypeStruct((B, Tpad), f32),
        jax.ShapeDtypeStruct((B, 1), f32),
        jax.ShapeDtypeStruct((B, D), f32),
    )
    out_specs = tuple(half_spec(s) for s in state_out_shapes)

    scratch = [
        pltpu.VMEM((BH, E), f32),
        pltpu.VMEM((BH, twoD), f32),
        pltpu.VMEM((BH, 1), f32),
        pltpu.VMEM((BH, 1), f32),
        pltpu.VMEM((BH, twoD), f32),
        pltpu.VMEM((BH, Tpad), f32),
    ]

    h, c, ctx, attn, attn_oov, p_gen, hidden = pl.pallas_call(
        _state_kernel,
        out_shape=state_out_shapes,
        grid=(2, nT),
        in_specs=in_specs,
        out_specs=out_specs,
        scratch_shapes=scratch,
        compiler_params=pltpu.CompilerParams(
            dimension_semantics=("parallel", "arbitrary"),
            vmem_limit_bytes=100 * 1024 * 1024),
    )(*state_inputs)

    vocab_padded = pl.pallas_call(
        _make_vocab_kernel(V),
        out_shape=jax.ShapeDtypeStruct((B, Wout), f32),
        grid=(2, nv),
        in_specs=[pl.BlockSpec((BH, D), lambda core, j: (core, 0)),
                  pl.BlockSpec((BH, 1), lambda core, j: (core, 0)),
                  pl.BlockSpec((D, TV), lambda core, j: (0, j)),
                  pl.BlockSpec((1, TV), lambda core, j: (0, j))],
        out_specs=pl.BlockSpec((BH, Wout), lambda core, j: (core, 0)),
        scratch_shapes=[pltpu.VMEM((BH, 1), f32)],
        compiler_params=pltpu.CompilerParams(
            dimension_semantics=("parallel", "arbitrary"),
            vmem_limit_bytes=100 * 1024 * 1024),
    )(hidden, p_gen, W2T, b2)

    vocab_dist_oov = vocab_padded[:, :V + n_extra]
    attn_b = attn[:, :T]
    attn_oov_b = attn_oov[:, :T]

    batch_idx = jnp.arange(B)[:, None]
    vocab_dist_final = vocab_dist_oov.at[batch_idx, encoder_input_extra_vocabs].add(attn_oov_b)

    decoder_hidden_next = (h[None], c[None])
    return vocab_dist_final, decoder_hidden_next, ctx, attn_b, None


# speedup vs baseline: 1.5196x; 1.5145x over previous
import jax
import jax.numpy as jnp
from jax.experimental import pallas as pl
from jax.experimental.pallas import tpu as pltpu


def _state_kernel(emb_ref, ctxprev_ref, h0_ref, c0_ref,
                  enc_ref, encf_ref, mask_ref,
                  wxt_ctx_ref, wxt_emb_ref, bx_ref,
                  wih4_ref, whh4_ref, b4_ref,
                  wst_h_ref, wst_c_ref, bs_ref, v_ref,
                  wp_ctx_ref, wp_h_ref, wp_c_ref, wp_x_ref, bp_ref,
                  w1t_h_ref, w1t_ctx_ref, b1_ref,
                  h_ref, c_ref, ctx_ref, attn_ref, attn_oov_ref,
                  pgen_ref, hidden_ref,
                  x_sc, decf_sc, m_sc, l_sc, ctxacc_sc, scores_sc):
    f32, bf16 = jnp.float32, jnp.bfloat16
    t = pl.program_id(1)
    n_t = pl.num_programs(1)
    TT = enc_ref.shape[1]

    def mm(a, w_ref):
        return jnp.dot(a.astype(bf16), w_ref[...], preferred_element_type=f32)

    @pl.when(t == 0)
    def _():
        x = (mm(ctxprev_ref[...], wxt_ctx_ref) + mm(emb_ref[...], wxt_emb_ref)
             + bx_ref[...])

        def gate(g):
            return (jnp.dot(x.astype(bf16), wih4_ref[g], preferred_element_type=f32)
                    + jnp.dot(h0_ref[...].astype(bf16), whh4_ref[g],
                              preferred_element_type=f32)
                    + b4_ref[g])

        i_g = jax.nn.sigmoid(gate(0))
        f_g = jax.nn.sigmoid(gate(1))
        g_g = jnp.tanh(gate(2))
        o_g = jax.nn.sigmoid(gate(3))
        c = f_g * c0_ref[...] + i_g * g_g
        h = o_g * jnp.tanh(c)
        h_ref[...] = h
        c_ref[...] = c
        x_sc[...] = x
        decf_sc[...] = mm(h, wst_h_ref) + mm(c, wst_c_ref) + bs_ref[...]
        m_sc[...] = jnp.full(m_sc.shape, -1e30, f32)
        l_sc[...] = jnp.zeros(l_sc.shape, f32)
        ctxacc_sc[...] = jnp.zeros(ctxacc_sc.shape, f32)

    start = pl.multiple_of(t * TT, TT)
    mask_t = mask_ref[:, pl.ds(start, TT)]
    energy = jnp.tanh(encf_ref[...].astype(f32) + decf_sc[...][:, None, :])
    scores = jnp.sum(energy * v_ref[...][None, :, :], axis=-1)
    scores = jnp.where(mask_t > 0.0, scores, -1e30)
    scores_sc[:, pl.ds(start, TT)] = scores

    m_old = m_sc[...]
    m_new = jnp.maximum(m_old, jnp.max(scores, axis=-1, keepdims=True))
    corr = jnp.exp(m_old - m_new)
    p = jnp.exp(scores - m_new)
    l_sc[...] = l_sc[...] * corr + jnp.sum(p, axis=-1, keepdims=True)
    ctxacc_sc[...] = (ctxacc_sc[...] * corr
                      + jnp.sum(p[:, :, None] * enc_ref[...].astype(f32), axis=1))
    m_sc[...] = m_new

    @pl.when(t == n_t - 1)
    def _():
        inv_l = 1.0 / jnp.maximum(l_sc[...], 1e-9)
        ctx = ctxacc_sc[...] * inv_l
        attn = jnp.exp(scores_sc[...] - m_sc[...]) * inv_l
        h = h_ref[...]
        c = c_ref[...]
        x = x_sc[...]
        p_lin = (jnp.sum(ctx * wp_ctx_ref[...], axis=-1, keepdims=True)
                 + jnp.sum(h * wp_h_ref[...], axis=-1, keepdims=True)
                 + jnp.sum(c * wp_c_ref[...], axis=-1, keepdims=True)
                 + jnp.sum(x * wp_x_ref[...], axis=-1, keepdims=True)
                 + bp_ref[...])
        p_gen = jax.nn.sigmoid(p_lin)
        ctx_ref[...] = ctx
        attn_ref[...] = attn
        attn_oov_ref[...] = (1.0 - p_gen) * attn
        pgen_ref[...] = p_gen
        hidden_ref[...] = mm(h, w1t_h_ref) + mm(ctx, w1t_ctx_ref) + b1_ref[...]


def _make_vocab_kernel(vocab_size):
    def _vocab_kernel(hidden_ref, pgen_ref, w2t_ref, b2_ref, out_ref, m_sc):
        f32, bf16 = jnp.float32, jnp.bfloat16
        j = pl.program_id(1)
        nv = pl.num_programs(1)
        TV = w2t_ref.shape[1]

        logits = (jnp.dot(hidden_ref[...].astype(bf16), w2t_ref[...],
                          preferred_element_type=f32) + b2_ref[...])

        @pl.when(j == 0)
        def _():
            m_sc[...] = jnp.full(m_sc.shape, -1e30, f32)

        m_sc[...] = jnp.maximum(m_sc[...], jnp.max(logits, axis=-1, keepdims=True))
        out_ref[:, pl.ds(pl.multiple_of(j * TV, TV), TV)] = logits

        @pl.when(j == nv - 1)
        def _():
            all_logits = out_ref[...]
            col = jax.lax.broadcasted_iota(jnp.int32, all_logits.shape, 1)
            probs = jnp.where(col < vocab_size,
                              jnp.exp(all_logits - m_sc[...]), 0.0)
            l = jnp.maximum(jnp.sum(probs, axis=-1, keepdims=True), 1e-30)
            out_ref[...] = probs * (pgen_ref[...] / l)

    return _vocab_kernel


def _make_scatter_kernel(rows_per_core, t_valid, n_q):
    def _scatter_kernel(val_ref, idx_ref, out_ref):
        f32, bf16, i32 = jnp.float32, jnp.bfloat16, jnp.int32
        Tpad = val_ref.shape[1]
        t_col = jax.lax.broadcasted_iota(i32, (1, Tpad), 1)
        q_iota = jax.lax.broadcasted_iota(i32, (n_q, Tpad), 0)
        r_iota = jax.lax.broadcasted_iota(i32, (128, Tpad), 0)

        def body(b, _):
            idx = idx_ref[pl.ds(b, 1), :]
            val = jnp.where(t_col < t_valid, val_ref[pl.ds(b, 1), :], 0.0)
            q = jax.lax.shift_right_logical(idx, 7)
            r = jax.lax.bitwise_and(idx, 127)
            Q = jnp.where(q_iota == q, 1.0, 0.0).astype(bf16)
            Ar = jnp.where(r_iota == r, val, 0.0).astype(bf16)
            R = jax.lax.dot_general(Q, Ar, (((1,), (1,)), ((), ())),
                                    preferred_element_type=f32)
            out_ref[pl.ds(b, 1)] = R[None]
            return 0

        jax.lax.fori_loop(0, rows_per_core, body, 0)

    return _scatter_kernel


def kernel(emb, WxT_ctx, WxT_emb, bx, Wih4T, Whh4T, b_lstm4, WsT_h, WsT_c, bs,
           v_att, Wp_ctx, Wp_h, Wp_c, Wp_x, bp, W1T_h, W1T_ctx, b1, W2T, b2,
           enc, encf, mask, decoder_input, h0, c0, previous_context_vector,
           extra_zeros, encoder_input_extra_vocabs):
    f32 = jnp.float32
    V, E = emb.shape
    B, Tpad, twoD = enc.shape
    D = twoD // 2
    T = encoder_input_extra_vocabs.shape[1]
    n_extra = extra_zeros.shape[1]
    Wout = b2.shape[1]
    TV = 4096
    nv = Wout // TV
    TT = 128
    nT = Tpad // TT
    BH = B // 2

    embedded = emb[decoder_input].astype(f32)
    ctxp = previous_context_vector.astype(f32)
    h0_2 = h0[0].astype(f32)
    c0_2 = c0[0].astype(f32)

    def half_spec(a):
        nd = a.ndim
        return pl.BlockSpec((BH,) + a.shape[1:],
                            lambda core, t, _nd=nd: (core,) + (0,) * (_nd - 1))

    def const_spec(a):
        nd = a.ndim
        return pl.BlockSpec(a.shape, lambda core, t, _nd=nd: (0,) * _nd)

    state_inputs = (
        embedded, ctxp, h0_2, c0_2,
        enc, encf, mask,
        WxT_ctx, WxT_emb, bx,
        Wih4T, Whh4T, b_lstm4,
        WsT_h, WsT_c, bs, v_att,
        Wp_ctx, Wp_h, Wp_c, Wp_x, bp,
        W1T_h, W1T_ctx, b1,
    )
    in_specs = ([half_spec(a) for a in state_inputs[:4]]
                + [pl.BlockSpec((BH, TT, twoD), lambda core, t: (core, t, 0)),
                   pl.BlockSpec((BH, TT, twoD), lambda core, t: (core, t, 0)),
                   half_spec(mask)]
                + [const_spec(a) for a in state_inputs[7:]])

    state_out_shapes = (
        jax.ShapeDtypeStruct((B, D), f32),
        jax.ShapeDtypeStruct((B, D), f32),
        jax.ShapeDtypeStruct((B, twoD), f32),
        jax.ShapeDtypeStruct((B, Tpad), f32),
        jax.ShapeDtypeStruct((B, Tpad), f32),
        jax.ShapeDtypeStruct((B, 1), f32),
        jax.ShapeDtypeStruct((B, D), f32),
    )
    out_specs = tuple(half_spec(s) for s in state_out_shapes)

    scratch = [
        pltpu.VMEM((BH, E), f32),
        pltpu.VMEM((BH, twoD), f32),
        pltpu.VMEM((BH, 1), f32),
        pltpu.VMEM((BH, 1), f32),
        pltpu.VMEM((BH, twoD), f32),
        pltpu.VMEM((BH, Tpad), f32),
    ]

    h, c, ctx, attn, attn_oov, p_gen, hidden = pl.pallas_call(
        _state_kernel,
        out_shape=state_out_shapes,
        grid=(2, nT),
        in_specs=in_specs,
        out_specs=out_specs,
        scratch_shapes=scratch,
        compiler_params=pltpu.CompilerParams(
            dimension_semantics=("parallel", "arbitrary"),
            vmem_limit_bytes=100 * 1024 * 1024),
    )(*state_inputs)

    vocab_padded = pl.pallas_call(
        _make_vocab_kernel(V),
        out_shape=jax.ShapeDtypeStruct((B, Wout), f32),
        grid=(2, nv),
        in_specs=[pl.BlockSpec((BH, D), lambda core, j: (core, 0)),
                  pl.BlockSpec((BH, 1), lambda core, j: (core, 0)),
                  pl.BlockSpec((D, TV), lambda core, j: (0, j)),
                  pl.BlockSpec((1, TV), lambda core, j: (0, j))],
        out_specs=pl.BlockSpec((BH, Wout), lambda core, j: (core, 0)),
        scratch_shapes=[pltpu.VMEM((BH, 1), f32)],
        compiler_params=pltpu.CompilerParams(
            dimension_semantics=("parallel", "arbitrary"),
            vmem_limit_bytes=100 * 1024 * 1024),
    )(hidden, p_gen, W2T, b2)

    n_q = (V + n_extra + 127) // 128
    idx_pad = jnp.pad(encoder_input_extra_vocabs.astype(jnp.int32),
                      ((0, 0), (0, Tpad - T)))
    scat = pl.pallas_call(
        _make_scatter_kernel(BH, T, n_q),
        out_shape=jax.ShapeDtypeStruct((B, n_q, 128), f32),
        grid=(2,),
        in_specs=[pl.BlockSpec((BH, Tpad), lambda core: (core, 0)),
                  pl.BlockSpec((BH, Tpad), lambda core: (core, 0))],
        out_specs=pl.BlockSpec((BH, n_q, 128), lambda core: (core, 0, 0)),
        compiler_params=pltpu.CompilerParams(
            dimension_semantics=("parallel",),
            vmem_limit_bytes=100 * 1024 * 1024),
    )(attn_oov, idx_pad)

    attn_b = attn[:, :T]
    vocab_dist_final = (vocab_padded[:, :V + n_extra]
                        + scat.reshape(B, n_q * 128)[:, :V + n_extra])

    decoder_hidden_next = (h[None], c[None])
    return vocab_dist_final, decoder_hidden_next, ctx, attn_b, None


# speedup vs baseline: 1.7215x; 1.1328x over previous
import jax
import jax.numpy as jnp
from jax.experimental import pallas as pl
from jax.experimental.pallas import tpu as pltpu


def _state_kernel(emb_ref, ctxprev_ref, h0_ref, c0_ref,
                  enc_ref, encf_ref, mask_ref,
                  wxt_ctx_ref, wxt_emb_ref, bx_ref,
                  wih4_ref, whh4_ref, b4_ref,
                  wst_h_ref, wst_c_ref, bs_ref, v_ref,
                  wp_ctx_ref, wp_h_ref, wp_c_ref, wp_x_ref, bp_ref,
                  w1t_h_ref, w1t_ctx_ref, b1_ref,
                  h_ref, c_ref, ctx_ref, attn_ref, attn_oov_ref,
                  pgen_ref, hidden_ref,
                  x_sc, decf_sc, m_sc, l_sc, ctxacc_sc, scores_sc):
    f32, bf16 = jnp.float32, jnp.bfloat16
    t = pl.program_id(1)
    n_t = pl.num_programs(1)
    TT = enc_ref.shape[1]

    def mm(a, w_ref):
        return jnp.dot(a.astype(bf16), w_ref[...], preferred_element_type=f32)

    @pl.when(t == 0)
    def _():
        x = (mm(ctxprev_ref[...], wxt_ctx_ref) + mm(emb_ref[...], wxt_emb_ref)
             + bx_ref[...])

        def gate(g):
            return (jnp.dot(x.astype(bf16), wih4_ref[g], preferred_element_type=f32)
                    + jnp.dot(h0_ref[...].astype(bf16), whh4_ref[g],
                              preferred_element_type=f32)
                    + b4_ref[g])

        i_g = jax.nn.sigmoid(gate(0))
        f_g = jax.nn.sigmoid(gate(1))
        g_g = jnp.tanh(gate(2))
        o_g = jax.nn.sigmoid(gate(3))
        c = f_g * c0_ref[...] + i_g * g_g
        h = o_g * jnp.tanh(c)
        h_ref[...] = h
        c_ref[...] = c
        x_sc[...] = x
        decf_sc[...] = mm(h, wst_h_ref) + mm(c, wst_c_ref) + bs_ref[...]
        m_sc[...] = jnp.full(m_sc.shape, -1e30, f32)
        l_sc[...] = jnp.zeros(l_sc.shape, f32)
        ctxacc_sc[...] = jnp.zeros(ctxacc_sc.shape, f32)

    start = pl.multiple_of(t * TT, TT)
    mask_t = mask_ref[:, pl.ds(start, TT)]
    energy = jnp.tanh(encf_ref[...].astype(f32) + decf_sc[...][:, None, :])
    scores = jnp.sum(energy * v_ref[...][None, :, :], axis=-1)
    scores = jnp.where(mask_t > 0.0, scores, -1e30)
    scores_sc[:, pl.ds(start, TT)] = scores

    m_old = m_sc[...]
    m_new = jnp.maximum(m_old, jnp.max(scores, axis=-1, keepdims=True))
    corr = jnp.exp(m_old - m_new)
    p = jnp.exp(scores - m_new)
    l_sc[...] = l_sc[...] * corr + jnp.sum(p, axis=-1, keepdims=True)
    ctxacc_sc[...] = (ctxacc_sc[...] * corr
                      + jnp.sum(p[:, :, None] * enc_ref[...].astype(f32), axis=1))
    m_sc[...] = m_new

    @pl.when(t == n_t - 1)
    def _():
        inv_l = 1.0 / jnp.maximum(l_sc[...], 1e-9)
        ctx = ctxacc_sc[...] * inv_l
        attn = jnp.exp(scores_sc[...] - m_sc[...]) * inv_l
        h = h_ref[...]
        c = c_ref[...]
        x = x_sc[...]
        p_lin = (jnp.sum(ctx * wp_ctx_ref[...], axis=-1, keepdims=True)
                 + jnp.sum(h * wp_h_ref[...], axis=-1, keepdims=True)
                 + jnp.sum(c * wp_c_ref[...], axis=-1, keepdims=True)
                 + jnp.sum(x * wp_x_ref[...], axis=-1, keepdims=True)
                 + bp_ref[...])
        p_gen = jax.nn.sigmoid(p_lin)
        ctx_ref[...] = ctx
        attn_ref[...] = attn
        attn_oov_ref[...] = (1.0 - p_gen) * attn
        pgen_ref[...] = p_gen
        hidden_ref[...] = mm(h, w1t_h_ref) + mm(ctx, w1t_ctx_ref) + b1_ref[...]


def _make_vocab_kernel(vocab_size):
    def _vocab_kernel(hidden_ref, pgen_ref, w2t_ref, b2_ref, scat_ref,
                      out_ref, m_sc):
        f32, bf16 = jnp.float32, jnp.bfloat16
        j = pl.program_id(1)
        nv = pl.num_programs(1)
        TV = w2t_ref.shape[1]

        logits = (jnp.dot(hidden_ref[...].astype(bf16), w2t_ref[...],
                          preferred_element_type=f32) + b2_ref[...])

        @pl.when(j == 0)
        def _():
            m_sc[...] = jnp.full(m_sc.shape, -1e30, f32)

        m_sc[...] = jnp.maximum(m_sc[...], jnp.max(logits, axis=-1, keepdims=True))
        out_ref[:, pl.ds(pl.multiple_of(j * TV, TV), TV)] = logits

        @pl.when(j == nv - 1)
        def _():
            all_logits = out_ref[...]
            col = jax.lax.broadcasted_iota(jnp.int32, all_logits.shape, 1)
            probs = jnp.where(col < vocab_size,
                              jnp.exp(all_logits - m_sc[...]), 0.0)
            l = jnp.maximum(jnp.sum(probs, axis=-1, keepdims=True), 1e-30)
            out_ref[...] = probs * (pgen_ref[...] / l)
            bh, n_q, _ = scat_ref.shape
            flat = scat_ref[...].reshape(bh, n_q * 128)
            out_ref[:, :n_q * 128] = out_ref[:, :n_q * 128] + flat

    return _vocab_kernel


def _make_scatter_kernel(rows_per_core, t_valid, n_q):
    def _scatter_kernel(val_ref, idx_ref, out_ref):
        f32, bf16, i32 = jnp.float32, jnp.bfloat16, jnp.int32
        Tpad = val_ref.shape[1]
        t_col = jax.lax.broadcasted_iota(i32, (1, Tpad), 1)
        q_iota = jax.lax.broadcasted_iota(i32, (n_q, Tpad), 0)
        r_iota = jax.lax.broadcasted_iota(i32, (128, Tpad), 0)

        def body(b, _):
            idx = idx_ref[pl.ds(b, 1), :]
            val = jnp.where(t_col < t_valid, val_ref[pl.ds(b, 1), :], 0.0)
            q = jax.lax.shift_right_logical(idx, 7)
            r = jax.lax.bitwise_and(idx, 127)
            Q = jnp.where(q_iota == q, 1.0, 0.0).astype(bf16)
            Ar = jnp.where(r_iota == r, val, 0.0).astype(bf16)
            R = jax.lax.dot_general(Q, Ar, (((1,), (1,)), ((), ())),
                                    preferred_element_type=f32)
            out_ref[pl.ds(b, 1)] = R[None]
            return 0

        jax.lax.fori_loop(0, rows_per_core, body, 0)

    return _scatter_kernel


def kernel(emb, WxT_ctx, WxT_emb, bx, Wih4T, Whh4T, b_lstm4, WsT_h, WsT_c, bs,
           v_att, Wp_ctx, Wp_h, Wp_c, Wp_x, bp, W1T_h, W1T_ctx, b1, W2T, b2,
           enc, encf, mask, decoder_input, h0, c0, previous_context_vector,
           extra_zeros, encoder_input_extra_vocabs):
    f32 = jnp.float32
    V, E = emb.shape
    B, Tpad, twoD = enc.shape
    D = twoD // 2
    T = encoder_input_extra_vocabs.shape[1]
    n_extra = extra_zeros.shape[1]
    Wout = b2.shape[1]
    TV = 4096
    nv = Wout // TV
    TT = 128
    nT = Tpad // TT
    BH = B // 2

    embedded = emb[decoder_input].astype(f32)
    ctxp = previous_context_vector.astype(f32)
    h0_2 = h0[0].astype(f32)
    c0_2 = c0[0].astype(f32)

    def half_spec(a):
        nd = a.ndim
        return pl.BlockSpec((BH,) + a.shape[1:],
                            lambda core, t, _nd=nd: (core,) + (0,) * (_nd - 1))

    def const_spec(a):
        nd = a.ndim
        return pl.BlockSpec(a.shape, lambda core, t, _nd=nd: (0,) * _nd)

    state_inputs = (
        embedded, ctxp, h0_2, c0_2,
        enc, encf, mask,
        WxT_ctx, WxT_emb, bx,
        Wih4T, Whh4T, b_lstm4,
        WsT_h, WsT_c, bs, v_att,
        Wp_ctx, Wp_h, Wp_c, Wp_x, bp,
        W1T_h, W1T_ctx, b1,
    )
    in_specs = ([half_spec(a) for a in state_inputs[:4]]
                + [pl.BlockSpec((BH, TT, twoD), lambda core, t: (core, t, 0)),
                   pl.BlockSpec((BH, TT, twoD), lambda core, t: (core, t, 0)),
                   half_spec(mask)]
                + [const_spec(a) for a in state_inputs[7:]])

    state_out_shapes = (
        jax.ShapeDtypeStruct((B, D), f32),
        jax.ShapeDtypeStruct((B, D), f32),
        jax.ShapeDtypeStruct((B, twoD), f32),
        jax.ShapeDtypeStruct((B, Tpad), f32),
        jax.ShapeDtypeStruct((B, Tpad), f32),
        jax.ShapeDtypeStruct((B, 1), f32),
        jax.ShapeDtypeStruct((B, D), f32),
    )
    out_specs = tuple(half_spec(s) for s in state_out_shapes)

    scratch = [
        pltpu.VMEM((BH, E), f32),
        pltpu.VMEM((BH, twoD), f32),
        pltpu.VMEM((BH, 1), f32),
        pltpu.VMEM((BH, 1), f32),
        pltpu.VMEM((BH, twoD), f32),
        pltpu.VMEM((BH, Tpad), f32),
    ]

    h, c, ctx, attn, attn_oov, p_gen, hidden = pl.pallas_call(
        _state_kernel,
        out_shape=state_out_shapes,
        grid=(2, nT),
        in_specs=in_specs,
        out_specs=out_specs,
        scratch_shapes=scratch,
        compiler_params=pltpu.CompilerParams(
            dimension_semantics=("parallel", "arbitrary"),
            vmem_limit_bytes=100 * 1024 * 1024),
    )(*state_inputs)

    n_q = (V + n_extra + 127) // 128
    idx_pad = jnp.pad(encoder_input_extra_vocabs.astype(jnp.int32),
                      ((0, 0), (0, Tpad - T)))
    scat = pl.pallas_call(
        _make_scatter_kernel(BH, T, n_q),
        out_shape=jax.ShapeDtypeStruct((B, n_q, 128), f32),
        grid=(2,),
        in_specs=[pl.BlockSpec((BH, Tpad), lambda core: (core, 0)),
                  pl.BlockSpec((BH, Tpad), lambda core: (core, 0))],
        out_specs=pl.BlockSpec((BH, n_q, 128), lambda core: (core, 0, 0)),
        compiler_params=pltpu.CompilerParams(
            dimension_semantics=("parallel",),
            vmem_limit_bytes=100 * 1024 * 1024),
    )(attn_oov, idx_pad)

    vocab_padded = pl.pallas_call(
        _make_vocab_kernel(V),
        out_shape=jax.ShapeDtypeStruct((B, Wout), f32),
        grid=(2, nv),
        in_specs=[pl.BlockSpec((BH, D), lambda core, j: (core, 0)),
                  pl.BlockSpec((BH, 1), lambda core, j: (core, 0)),
                  pl.BlockSpec((D, TV), lambda core, j: (0, j)),
                  pl.BlockSpec((1, TV), lambda core, j: (0, j)),
                  pl.BlockSpec((BH, n_q, 128), lambda core, j: (core, 0, 0))],
        out_specs=pl.BlockSpec((BH, Wout), lambda core, j: (core, 0)),
        scratch_shapes=[pltpu.VMEM((BH, 1), f32)],
        compiler_params=pltpu.CompilerParams(
            dimension_semantics=("parallel", "arbitrary"),
            vmem_limit_bytes=100 * 1024 * 1024),
    )(hidden, p_gen, W2T, b2, scat)

    attn_b = attn[:, :T]
    vocab_dist_final = vocab_padded[:, :V + n_extra]

    decoder_hidden_next = (h[None], c[None])
    return vocab_dist_final, decoder_hidden_next, ctx, attn_b, None


# speedup vs baseline: 2.5755x; 1.4961x over previous
import jax
import jax.numpy as jnp
from jax.experimental import pallas as pl
from jax.experimental.pallas import tpu as pltpu


def _make_state_kernel(rows_per_core):
    def _state_kernel(dec_ref, emb_ref, ctxprev_ref, h0_ref, c0_ref,
                      enc_ref, encf_ref, mask_ref,
                      wxt_ctx_ref, wxt_emb_ref, bx_ref,
                      wih4_ref, whh4_ref, b4_ref,
                      wst_h_ref, wst_c_ref, bs_ref, v_ref,
                      wp_ctx_ref, wp_h_ref, wp_c_ref, wp_x_ref, bp_ref,
                      w1t_h_ref, w1t_ctx_ref, b1_ref,
                      h_ref, c_ref, ctx_ref, attn_ref, attn_oov_ref,
                      pgen_ref, hidden_ref,
                      emb_sc, x_sc, decf_sc, m_sc, l_sc, ctxacc_sc, scores_sc,
                      emb_sem):
        f32, bf16 = jnp.float32, jnp.bfloat16
        core = pl.program_id(0)
        t = pl.program_id(1)
        n_t = pl.num_programs(1)
        TT = enc_ref.shape[1]

        def mm(a, w_ref):
            return jnp.dot(a.astype(bf16), w_ref[...], preferred_element_type=f32)

        @pl.when(t == 0)
        def _():
            base = core * rows_per_core

            def start_row(b, _):
                row = dec_ref[base + b]
                pltpu.make_async_copy(
                    emb_ref.at[pl.ds(row, 1), :],
                    emb_sc.at[pl.ds(b, 1), :], emb_sem).start()
                return 0

            jax.lax.fori_loop(0, rows_per_core, start_row, 0)

            def wait_row(b, _):
                pltpu.make_async_copy(
                    emb_ref.at[pl.ds(0, 1), :],
                    emb_sc.at[pl.ds(b, 1), :], emb_sem).wait()
                return 0

            jax.lax.fori_loop(0, rows_per_core, wait_row, 0)

            x = (mm(ctxprev_ref[...], wxt_ctx_ref) + mm(emb_sc[...], wxt_emb_ref)
                 + bx_ref[...])

            def gate(g):
                return (jnp.dot(x.astype(bf16), wih4_ref[g],
                                preferred_element_type=f32)
                        + jnp.dot(h0_ref[...].astype(bf16), whh4_ref[g],
                                  preferred_element_type=f32)
                        + b4_ref[g])

            i_g = jax.nn.sigmoid(gate(0))
            f_g = jax.nn.sigmoid(gate(1))
            g_g = jnp.tanh(gate(2))
            o_g = jax.nn.sigmoid(gate(3))
            c = f_g * c0_ref[...] + i_g * g_g
            h = o_g * jnp.tanh(c)
            h_ref[...] = h
            c_ref[...] = c
            x_sc[...] = x
            decf_sc[...] = mm(h, wst_h_ref) + mm(c, wst_c_ref) + bs_ref[...]
            m_sc[...] = jnp.full(m_sc.shape, -1e30, f32)
            l_sc[...] = jnp.zeros(l_sc.shape, f32)
            ctxacc_sc[...] = jnp.zeros(ctxacc_sc.shape, f32)

        start = pl.multiple_of(t * TT, TT)
        mask_t = mask_ref[:, pl.ds(start, TT)]
        energy = jnp.tanh(encf_ref[...].astype(f32) + decf_sc[...][:, None, :])
        scores = jnp.sum(energy * v_ref[...][None, :, :], axis=-1)
        scores = jnp.where(mask_t > 0.0, scores, -1e30)
        scores_sc[:, pl.ds(start, TT)] = scores

        m_old = m_sc[...]
        m_new = jnp.maximum(m_old, jnp.max(scores, axis=-1, keepdims=True))
        corr = jnp.exp(m_old - m_new)
        p = jnp.exp(scores - m_new)
        l_sc[...] = l_sc[...] * corr + jnp.sum(p, axis=-1, keepdims=True)
        ctxacc_sc[...] = (ctxacc_sc[...] * corr
                          + jnp.sum(p[:, :, None] * enc_ref[...].astype(f32), axis=1))
        m_sc[...] = m_new

        @pl.when(t == n_t - 1)
        def _():
            inv_l = 1.0 / jnp.maximum(l_sc[...], 1e-9)
            ctx = ctxacc_sc[...] * inv_l
            attn = jnp.exp(scores_sc[...] - m_sc[...]) * inv_l
            h = h_ref[...]
            c = c_ref[...]
            x = x_sc[...]
            p_lin = (jnp.sum(ctx * wp_ctx_ref[...], axis=-1, keepdims=True)
                     + jnp.sum(h * wp_h_ref[...], axis=-1, keepdims=True)
                     + jnp.sum(c * wp_c_ref[...], axis=-1, keepdims=True)
                     + jnp.sum(x * wp_x_ref[...], axis=-1, keepdims=True)
                     + bp_ref[...])
            p_gen = jax.nn.sigmoid(p_lin)
            ctx_ref[...] = ctx
            attn_ref[...] = attn
            attn_oov_ref[...] = (1.0 - p_gen) * attn
            pgen_ref[...] = p_gen
            hidden_ref[...] = mm(h, w1t_h_ref) + mm(ctx, w1t_ctx_ref) + b1_ref[...]

    return _state_kernel


def _make_vocab_kernel(vocab_size, nv, rows_per_core, t_valid, n_q):
    def _vocab_kernel(hidden_ref, pgen_ref, w2t_ref, b2_ref, val_ref, idx_ref,
                      out_ref, logits_sc, m_sc, l_sc, scat_sc):
        f32, bf16, i32 = jnp.float32, jnp.bfloat16, jnp.int32
        p_id = pl.program_id(1)
        TV = w2t_ref.shape[1]
        Tpad = val_ref.shape[1]
        rows_per_phase = rows_per_core // nv
        q_per_slab = n_q // nv

        @pl.when(p_id < nv)
        def _():
            j = p_id
            logits = (jnp.dot(hidden_ref[...].astype(bf16), w2t_ref[...],
                              preferred_element_type=f32) + b2_ref[...])

            @pl.when(j == 0)
            def _():
                m_sc[...] = jnp.full(m_sc.shape, -1e30, f32)
                l_sc[...] = jnp.zeros(l_sc.shape, f32)

            m_old = m_sc[...]
            m_new = jnp.maximum(m_old, jnp.max(logits, axis=-1, keepdims=True))
            l_sc[...] = (l_sc[...] * jnp.exp(m_old - m_new)
                         + jnp.sum(jnp.exp(logits - m_new), axis=-1, keepdims=True))
            m_sc[...] = m_new
            logits_sc[:, pl.ds(pl.multiple_of(j * TV, TV), TV)] = logits

            t_col = jax.lax.broadcasted_iota(i32, (1, Tpad), 1)
            q_iota = jax.lax.broadcasted_iota(i32, (n_q, Tpad), 0)
            r_iota = jax.lax.broadcasted_iota(i32, (128, Tpad), 0)
            for rr in range(rows_per_phase):
                b = j * rows_per_phase + rr
                idx = idx_ref[pl.ds(b, 1), :]
                val = jnp.where(t_col < t_valid, val_ref[pl.ds(b, 1), :], 0.0)
                q = jax.lax.shift_right_logical(idx, 7)
                r = jax.lax.bitwise_and(idx, 127)
                Q = jnp.where(q_iota == q, 1.0, 0.0).astype(bf16)
                Ar = jnp.where(r_iota == r, val, 0.0).astype(bf16)
                R = jax.lax.dot_general(Q, Ar, (((1,), (1,)), ((), ())),
                                        preferred_element_type=f32)
                scat_sc[pl.ds(b, 1)] = R[None]

        @pl.when(p_id >= nv)
        def _():
            f = p_id - nv
            seg = logits_sc[:, pl.ds(pl.multiple_of(f * TV, TV), TV)]
            col = (jax.lax.broadcasted_iota(i32, seg.shape, 1) + f * TV)
            scale = pgen_ref[...] / jnp.maximum(l_sc[...], 1e-30)
            probs = jnp.where(col < vocab_size,
                              jnp.exp(seg - m_sc[...]), 0.0) * scale
            flat = scat_sc[pl.ds(0, scat_sc.shape[0]),
                           pl.ds(f * q_per_slab, q_per_slab), :]
            out_ref[...] = probs + flat.reshape(probs.shape)

    return _vocab_kernel


def kernel(emb, WxT_ctx, WxT_emb, bx, Wih4T, Whh4T, b_lstm4, WsT_h, WsT_c, bs,
           v_att, Wp_ctx, Wp_h, Wp_c, Wp_x, bp, W1T_h, W1T_ctx, b1, W2T, b2,
           enc, encf, mask, decoder_input, h0, c0, previous_context_vector,
           extra_zeros, encoder_input_extra_vocabs):
    f32 = jnp.float32
    V, E = emb.shape
    B, Tpad, twoD = enc.shape
    D = twoD // 2
    T = encoder_input_extra_vocabs.shape[1]
    n_extra = extra_zeros.shape[1]
    Vext = V + n_extra
    n_q = (Vext + 127) // 128
    nv = 8
    TV = (n_q * 128) // nv
    TT = 128
    nT = Tpad // TT
    BH = B // 2

    ctxp = previous_context_vector.astype(f32)
    h0_2 = h0[0].astype(f32)
    c0_2 = c0[0].astype(f32)
    dec_i = decoder_input.astype(jnp.int32)

    def half_spec(a):
        nd = a.ndim
        return pl.BlockSpec((BH,) + a.shape[1:],
                            lambda core, t, _nd=nd: (core,) + (0,) * (_nd - 1))

    def const_spec(a):
        nd = a.ndim
        return pl.BlockSpec(a.shape, lambda core, t, _nd=nd: (0,) * _nd)

    state_inputs = (
        dec_i, emb, ctxp, h0_2, c0_2,
        enc, encf, mask,
        WxT_ctx, WxT_emb, bx,
        Wih4T, Whh4T, b_lstm4,
        WsT_h, WsT_c, bs, v_att,
        Wp_ctx, Wp_h, Wp_c, Wp_x, bp,
        W1T_h, W1T_ctx, b1,
    )
    in_specs = ([pl.BlockSpec(memory_space=pltpu.SMEM),
                 pl.BlockSpec(memory_space=pl.ANY)]
                + [half_spec(a) for a in state_inputs[2:5]]
                + [pl.BlockSpec((BH, TT, twoD), lambda core, t: (core, t, 0)),
                   pl.BlockSpec((BH, TT, twoD), lambda core, t: (core, t, 0)),
                   half_spec(mask)]
                + [const_spec(a) for a in state_inputs[8:]])

    state_out_shapes = (
        jax.ShapeDtypeStruct((B, D), f32),
        jax.ShapeDtypeStruct((B, D), f32),
        jax.ShapeDtypeStruct((B, twoD), f32),
        jax.ShapeDtypeStruct((B, Tpad), f32),
        jax.ShapeDtypeStruct((B, Tpad), f32),
        jax.ShapeDtypeStruct((B, 1), f32),
        jax.ShapeDtypeStruct((B, D), f32),
    )
    out_specs = tuple(half_spec(s) for s in state_out_shapes)

    scratch = [
        pltpu.VMEM((BH, E), f32),
        pltpu.VMEM((BH, E), f32),
        pltpu.VMEM((BH, twoD), f32),
        pltpu.VMEM((BH, 1), f32),
        pltpu.VMEM((BH, 1), f32),
        pltpu.VMEM((BH, twoD), f32),
        pltpu.VMEM((BH, Tpad), f32),
        pltpu.SemaphoreType.DMA,
    ]

    h, c, ctx, attn, attn_oov, p_gen, hidden = pl.pallas_call(
        _make_state_kernel(BH),
        out_shape=state_out_shapes,
        grid=(2, nT),
        in_specs=in_specs,
        out_specs=out_specs,
        scratch_shapes=scratch,
        compiler_params=pltpu.CompilerParams(
            dimension_semantics=("parallel", "arbitrary"),
            vmem_limit_bytes=100 * 1024 * 1024),
    )(*state_inputs)

    idx_pad = jnp.pad(encoder_input_extra_vocabs.astype(jnp.int32),
                      ((0, 0), (0, Tpad - T)))

    vocab_dist_final = pl.pallas_call(
        _make_vocab_kernel(V, nv, BH, T, n_q),
        out_shape=jax.ShapeDtypeStruct((B, Vext), f32),
        grid=(2, 2 * nv),
        in_specs=[pl.BlockSpec((BH, D), lambda core, p: (core, 0)),
                  pl.BlockSpec((BH, 1), lambda core, p: (core, 0)),
                  pl.BlockSpec((D, TV),
                               lambda core, p: (0, jnp.minimum(p, nv - 1))),
                  pl.BlockSpec((1, TV),
                               lambda core, p: (0, jnp.minimum(p, nv - 1))),
                  pl.BlockSpec((BH, Tpad), lambda core, p: (core, 0)),
                  pl.BlockSpec((BH, Tpad), lambda core, p: (core, 0))],
        out_specs=pl.BlockSpec((BH, TV),
                               lambda core, p: (core, jnp.maximum(p - nv, 0))),
        scratch_shapes=[pltpu.VMEM((BH, nv * TV), f32),
                        pltpu.VMEM((BH, 1), f32),
                        pltpu.VMEM((BH, 1), f32),
                        pltpu.VMEM((BH, n_q, 128), f32)],
        compiler_params=pltpu.CompilerParams(
            dimension_semantics=("parallel", "arbitrary"),
            vmem_limit_bytes=100 * 1024 * 1024),
    )(hidden, p_gen, W2T, b2, attn_oov, idx_pad)

    attn_b = attn[:, :T]
    decoder_hidden_next = (h[None], c[None])
    return vocab_dist_final, decoder_hidden_next, ctx, attn_b, None


# speedup vs baseline: 2.6786x; 1.0400x over previous
import jax
import jax.numpy as jnp
from jax.experimental import pallas as pl
from jax.experimental.pallas import tpu as pltpu


def _make_state_kernel(rows_per_core):
    def _state_kernel(dec_ref, emb_ref, ctxprev_ref, h0_ref, c0_ref,
                      enc_ref, encf_ref, mask_ref,
                      wxt_ctx_ref, wxt_emb_ref, bx_ref,
                      wih4_ref, whh4_ref, b4_ref,
                      wst_h_ref, wst_c_ref, bs_ref, v_ref,
                      wp_ctx_ref, wp_h_ref, wp_c_ref, wp_x_ref, bp_ref,
                      w1t_h_ref, w1t_ctx_ref, b1_ref,
                      h_ref, c_ref, ctx_ref, attn_ref, attn_oov_ref,
                      pgen_ref, hidden_ref,
                      emb_sc, x_sc, decf_sc, m_sc, l_sc, ctxacc_sc, scores_sc,
                      emb_sem):
        f32, bf16 = jnp.float32, jnp.bfloat16
        core = pl.program_id(0)
        t = pl.program_id(1)
        n_t = pl.num_programs(1)
        TT = enc_ref.shape[1]

        def mm(a, w_ref):
            return jnp.dot(a.astype(bf16), w_ref[...], preferred_element_type=f32)

        @pl.when(t == 0)
        def _():
            base = core * rows_per_core

            def start_row(b, _):
                row = dec_ref[base + b]
                pltpu.make_async_copy(
                    emb_ref.at[pl.ds(row, 1), :],
                    emb_sc.at[pl.ds(b, 1), :], emb_sem).start()
                return 0

            jax.lax.fori_loop(0, rows_per_core, start_row, 0)

            x_part = mm(ctxprev_ref[...], wxt_ctx_ref) + bx_ref[...]

            def wait_row(b, _):
                pltpu.make_async_copy(
                    emb_ref.at[pl.ds(0, 1), :],
                    emb_sc.at[pl.ds(b, 1), :], emb_sem).wait()
                return 0

            jax.lax.fori_loop(0, rows_per_core, wait_row, 0)

            x = x_part + mm(emb_sc[...], wxt_emb_ref)

            def gate(g):
                return (jnp.dot(x.astype(bf16), wih4_ref[g],
                                preferred_element_type=f32)
                        + jnp.dot(h0_ref[...].astype(bf16), whh4_ref[g],
                                  preferred_element_type=f32)
                        + b4_ref[g])

            i_g = jax.nn.sigmoid(gate(0))
            f_g = jax.nn.sigmoid(gate(1))
            g_g = jnp.tanh(gate(2))
            o_g = jax.nn.sigmoid(gate(3))
            c = f_g * c0_ref[...] + i_g * g_g
            h = o_g * jnp.tanh(c)
            h_ref[...] = h
            c_ref[...] = c
            x_sc[...] = x
            decf_sc[...] = mm(h, wst_h_ref) + mm(c, wst_c_ref) + bs_ref[...]
            m_sc[...] = jnp.full(m_sc.shape, -1e30, f32)
            l_sc[...] = jnp.zeros(l_sc.shape, f32)
            ctxacc_sc[...] = jnp.zeros(ctxacc_sc.shape, f32)

        start = pl.multiple_of(t * TT, TT)
        mask_t = mask_ref[:, pl.ds(start, TT)]
        energy = jnp.tanh(encf_ref[...].astype(f32) + decf_sc[...][:, None, :])
        scores = jnp.sum(energy * v_ref[...][None, :, :], axis=-1)
        scores = jnp.where(mask_t > 0.0, scores, -1e30)
        scores_sc[:, pl.ds(start, TT)] = scores

        m_old = m_sc[...]
        m_new = jnp.maximum(m_old, jnp.max(scores, axis=-1, keepdims=True))
        corr = jnp.exp(m_old - m_new)
        p = jnp.exp(scores - m_new)
        l_sc[...] = l_sc[...] * corr + jnp.sum(p, axis=-1, keepdims=True)
        ctxacc_sc[...] = (ctxacc_sc[...] * corr
                          + jnp.sum(p[:, :, None] * enc_ref[...].astype(f32), axis=1))
        m_sc[...] = m_new

        @pl.when(t == n_t - 1)
        def _():
            inv_l = 1.0 / jnp.maximum(l_sc[...], 1e-9)
            ctx = ctxacc_sc[...] * inv_l
            attn = jnp.exp(scores_sc[...] - m_sc[...]) * inv_l
            h = h_ref[...]
            c = c_ref[...]
            x = x_sc[...]
            p_lin = (jnp.sum(ctx * wp_ctx_ref[...], axis=-1, keepdims=True)
                     + jnp.sum(h * wp_h_ref[...], axis=-1, keepdims=True)
                     + jnp.sum(c * wp_c_ref[...], axis=-1, keepdims=True)
                     + jnp.sum(x * wp_x_ref[...], axis=-1, keepdims=True)
                     + bp_ref[...])
            p_gen = jax.nn.sigmoid(p_lin)
            ctx_ref[...] = ctx
            attn_ref[...] = attn
            attn_oov_ref[...] = (1.0 - p_gen) * attn
            pgen_ref[...] = p_gen
            hidden_ref[...] = mm(h, w1t_h_ref) + mm(ctx, w1t_ctx_ref) + b1_ref[...]

    return _state_kernel


def _make_vocab_kernel(vocab_size, nv, rows_per_core, t_valid, n_q):
    def _vocab_kernel(hidden_ref, pgen_ref, w2t_ref, b2_ref, val_ref, idx_ref,
                      out_ref, logits_sc, m_sc, l_sc, scat_sc):
        f32, bf16, i32 = jnp.float32, jnp.bfloat16, jnp.int32
        p_id = pl.program_id(1)
        TV = w2t_ref.shape[1]
        Tpad = val_ref.shape[1]
        rows_per_phase = rows_per_core // nv
        q_per_slab = n_q // nv

        @pl.when(p_id < nv)
        def _():
            j = p_id
            logits = (jnp.dot(hidden_ref[...].astype(bf16), w2t_ref[...],
                              preferred_element_type=f32) + b2_ref[...])

            @pl.when(j == 0)
            def _():
                m_sc[...] = jnp.full(m_sc.shape, -1e30, f32)
                l_sc[...] = jnp.zeros(l_sc.shape, f32)

            m_old = m_sc[...]
            m_new = jnp.maximum(m_old, jnp.max(logits, axis=-1, keepdims=True))
            l_sc[...] = (l_sc[...] * jnp.exp(m_old - m_new)
                         + jnp.sum(jnp.exp(logits - m_new), axis=-1, keepdims=True))
            m_sc[...] = m_new
            logits_sc[:, pl.ds(pl.multiple_of(j * TV, TV), TV)] = logits

            t_col = jax.lax.broadcasted_iota(i32, (1, Tpad), 1)
            q_iota = jax.lax.broadcasted_iota(i32, (n_q, Tpad), 0)
            r_iota = jax.lax.broadcasted_iota(i32, (128, Tpad), 0)
            for rr in range(rows_per_phase):
                b = j * rows_per_phase + rr
                idx = idx_ref[pl.ds(b, 1), :]
                val = jnp.where(t_col < t_valid, val_ref[pl.ds(b, 1), :], 0.0)
                q = jax.lax.shift_right_logical(idx, 7)
                r = jax.lax.bitwise_and(idx, 127)
                Q = jnp.where(q_iota == q, 1.0, 0.0).astype(bf16)
                Ar = jnp.where(r_iota == r, val, 0.0).astype(bf16)
                R = jax.lax.dot_general(Q, Ar, (((1,), (1,)), ((), ())),
                                        preferred_element_type=f32)
                scat_sc[pl.ds(b, 1)] = R[None]

        @pl.when(p_id >= nv)
        def _():
            f = p_id - nv
            seg = logits_sc[:, pl.ds(pl.multiple_of(f * TV, TV), TV)]
            col = (jax.lax.broadcasted_iota(i32, seg.shape, 1) + f * TV)
            scale = pgen_ref[...] / jnp.maximum(l_sc[...], 1e-30)
            probs = jnp.where(col < vocab_size,
                              jnp.exp(seg - m_sc[...]), 0.0) * scale
            flat = scat_sc[pl.ds(0, scat_sc.shape[0]),
                           pl.ds(f * q_per_slab, q_per_slab), :]
            out_ref[...] = probs + flat.reshape(probs.shape)

    return _vocab_kernel


def kernel(emb, WxT_ctx, WxT_emb, bx, Wih4T, Whh4T, b_lstm4, WsT_h, WsT_c, bs,
           v_att, Wp_ctx, Wp_h, Wp_c, Wp_x, bp, W1T_h, W1T_ctx, b1, W2T, b2,
           enc, encf, mask, decoder_input, h0, c0, previous_context_vector,
           extra_zeros, encoder_input_extra_vocabs):
    f32 = jnp.float32
    V, E = emb.shape
    B, Tpad, twoD = enc.shape
    D = twoD // 2
    T = encoder_input_extra_vocabs.shape[1]
    n_extra = extra_zeros.shape[1]
    Vext = V + n_extra
    n_q = (Vext + 127) // 128
    nv = 4
    TV = (n_q * 128) // nv
    TT = 256
    nT = Tpad // TT
    BH = B // 2

    ctxp = previous_context_vector.astype(f32)
    h0_2 = h0[0].astype(f32)
    c0_2 = c0[0].astype(f32)
    dec_i = decoder_input.astype(jnp.int32)

    def half_spec(a):
        nd = a.ndim
        return pl.BlockSpec((BH,) + a.shape[1:],
                            lambda core, t, _nd=nd: (core,) + (0,) * (_nd - 1))

    def const_spec(a):
        nd = a.ndim
        return pl.BlockSpec(a.shape, lambda core, t, _nd=nd: (0,) * _nd)

    state_inputs = (
        dec_i, emb, ctxp, h0_2, c0_2,
        enc, encf, mask,
        WxT_ctx, WxT_emb, bx,
        Wih4T, Whh4T, b_lstm4,
        WsT_h, WsT_c, bs, v_att,
        Wp_ctx, Wp_h, Wp_c, Wp_x, bp,
        W1T_h, W1T_ctx, b1,
    )
    in_specs = ([pl.BlockSpec(memory_space=pltpu.SMEM),
                 pl.BlockSpec(memory_space=pl.ANY)]
                + [half_spec(a) for a in state_inputs[2:5]]
                + [pl.BlockSpec((BH, TT, twoD), lambda core, t: (core, t, 0)),
                   pl.BlockSpec((BH, TT, twoD), lambda core, t: (core, t, 0)),
                   half_spec(mask)]
                + [const_spec(a) for a in state_inputs[8:]])

    state_out_shapes = (
        jax.ShapeDtypeStruct((B, D), f32),
        jax.ShapeDtypeStruct((B, D), f32),
        jax.ShapeDtypeStruct((B, twoD), f32),
        jax.ShapeDtypeStruct((B, Tpad), f32),
        jax.ShapeDtypeStruct((B, Tpad), f32),
        jax.ShapeDtypeStruct((B, 1), f32),
        jax.ShapeDtypeStruct((B, D), f32),
    )
    out_specs = tuple(half_spec(s) for s in state_out_shapes)

    scratch = [
        pltpu.VMEM((BH, E), f32),
        pltpu.VMEM((BH, E), f32),
        pltpu.VMEM((BH, twoD), f32),
        pltpu.VMEM((BH, 1), f32),
        pltpu.VMEM((BH, 1), f32),
        pltpu.VMEM((BH, twoD), f32),
        pltpu.VMEM((BH, Tpad), f32),
        pltpu.SemaphoreType.DMA,
    ]

    h, c, ctx, attn, attn_oov, p_gen, hidden = pl.pallas_call(
        _make_state_kernel(BH),
        out_shape=state_out_shapes,
        grid=(2, nT),
        in_specs=in_specs,
        out_specs=out_specs,
        scratch_shapes=scratch,
        compiler_params=pltpu.CompilerParams(
            dimension_semantics=("parallel", "arbitrary"),
            vmem_limit_bytes=100 * 1024 * 1024),
    )(*state_inputs)

    idx_pad = jnp.pad(encoder_input_extra_vocabs.astype(jnp.int32),
                      ((0, 0), (0, Tpad - T)))

    vocab_dist_final = pl.pallas_call(
        _make_vocab_kernel(V, nv, BH, T, n_q),
        out_shape=jax.ShapeDtypeStruct((B, Vext), f32),
        grid=(2, 2 * nv),
        in_specs=[pl.BlockSpec((BH, D), lambda core, p: (core, 0)),
                  pl.BlockSpec((BH, 1), lambda core, p: (core, 0)),
                  pl.BlockSpec((D, TV),
                               lambda core, p: (0, jnp.minimum(p, nv - 1))),
                  pl.BlockSpec((1, TV),
                               lambda core, p: (0, jnp.minimum(p, nv - 1))),
                  pl.BlockSpec((BH, Tpad), lambda core, p: (core, 0)),
                  pl.BlockSpec((BH, Tpad), lambda core, p: (core, 0))],
        out_specs=pl.BlockSpec((BH, TV),
                               lambda core, p: (core, jnp.maximum(p - nv, 0))),
        scratch_shapes=[pltpu.VMEM((BH, nv * TV), f32),
                        pltpu.VMEM((BH, 1), f32),
                        pltpu.VMEM((BH, 1), f32),
                        pltpu.VMEM((BH, n_q, 128), f32)],
        compiler_params=pltpu.CompilerParams(
            dimension_semantics=("parallel", "arbitrary"),
            vmem_limit_bytes=100 * 1024 * 1024),
    )(hidden, p_gen, W2T, b2, attn_oov, idx_pad)

    attn_b = attn[:, :T]
    decoder_hidden_next = (h[None], c[None])
    return vocab_dist_final, decoder_hidden_next, ctx, attn_b, None


# speedup vs baseline: 2.7139x; 1.0132x over previous
import jax
import jax.numpy as jnp
from jax.experimental import pallas as pl
from jax.experimental.pallas import tpu as pltpu


def _make_state_kernel(rows_per_core):
    def _state_kernel(dec_ref, emb_ref, ctxprev_ref, h0_ref, c0_ref,
                      enc_ref, encf_ref, mask_ref,
                      wxt_ctx_ref, wxt_emb_ref, bx_ref,
                      wih4_ref, whh4_ref, b4_ref,
                      wst_h_ref, wst_c_ref, bs_ref, v_ref,
                      wp_ctx_ref, wp_h_ref, wp_c_ref, wp_x_ref, bp_ref,
                      w1t_h_ref, w1t_ctx_ref, b1_ref,
                      h_ref, c_ref, ctx_ref, attn_ref, attn_oov_ref,
                      pgen_ref, hidden_ref,
                      emb_sc, x_sc, decf_sc, m_sc, l_sc, ctxacc_sc, scores_sc,
                      emb_sem):
        f32, bf16 = jnp.float32, jnp.bfloat16
        core = pl.program_id(0)
        t = pl.program_id(1)
        n_t = pl.num_programs(1)
        TT = enc_ref.shape[1]

        def mm(a, w_ref):
            return jnp.dot(a.astype(bf16), w_ref[...], preferred_element_type=f32)

        @pl.when(t == 0)
        def _():
            base = core * rows_per_core

            def start_row(b, _):
                row = dec_ref[base + b]
                pltpu.make_async_copy(
                    emb_ref.at[pl.ds(row, 1), :],
                    emb_sc.at[pl.ds(b, 1), :], emb_sem).start()
                return 0

            jax.lax.fori_loop(0, rows_per_core, start_row, 0)

            x_part = mm(ctxprev_ref[...], wxt_ctx_ref) + bx_ref[...]

            def wait_row(b, _):
                pltpu.make_async_copy(
                    emb_ref.at[pl.ds(0, 1), :],
                    emb_sc.at[pl.ds(b, 1), :], emb_sem).wait()
                return 0

            jax.lax.fori_loop(0, rows_per_core, wait_row, 0)

            x = x_part + mm(emb_sc[...], wxt_emb_ref)

            def gate(g):
                return (jnp.dot(x.astype(bf16), wih4_ref[g],
                                preferred_element_type=f32)
                        + jnp.dot(h0_ref[...].astype(bf16), whh4_ref[g],
                                  preferred_element_type=f32)
                        + b4_ref[g])

            i_g = jax.nn.sigmoid(gate(0))
            f_g = jax.nn.sigmoid(gate(1))
            g_g = jnp.tanh(gate(2))
            o_g = jax.nn.sigmoid(gate(3))
            c = f_g * c0_ref[...] + i_g * g_g
            h = o_g * jnp.tanh(c)
            h_ref[...] = h
            c_ref[...] = c
            x_sc[...] = x
            decf_sc[...] = mm(h, wst_h_ref) + mm(c, wst_c_ref) + bs_ref[...]
            m_sc[...] = jnp.full(m_sc.shape, -1e30, f32)
            l_sc[...] = jnp.zeros(l_sc.shape, f32)
            ctxacc_sc[...] = jnp.zeros(ctxacc_sc.shape, f32)

        start = pl.multiple_of(t * TT, TT)
        mask_t = mask_ref[:, pl.ds(start, TT)]
        decf_bf = decf_sc[...].astype(bf16)
        v_bf = v_ref[...].astype(bf16)
        energy = jnp.tanh(encf_ref[...] + decf_bf[:, None, :])
        scores = jnp.sum((energy * v_bf[None, :, :]).astype(f32), axis=-1)
        scores = jnp.where(mask_t > 0.0, scores, -1e30)
        scores_sc[:, pl.ds(start, TT)] = scores

        m_old = m_sc[...]
        m_new = jnp.maximum(m_old, jnp.max(scores, axis=-1, keepdims=True))
        corr = jnp.exp(m_old - m_new)
        p = jnp.exp(scores - m_new)
        l_sc[...] = l_sc[...] * corr + jnp.sum(p, axis=-1, keepdims=True)
        ctxacc_sc[...] = (ctxacc_sc[...] * corr
                          + jnp.sum((p.astype(bf16)[:, :, None]
                                     * enc_ref[...]).astype(f32), axis=1))
        m_sc[...] = m_new

        @pl.when(t == n_t - 1)
        def _():
            inv_l = 1.0 / jnp.maximum(l_sc[...], 1e-9)
            ctx = ctxacc_sc[...] * inv_l
            attn = jnp.exp(scores_sc[...] - m_sc[...]) * inv_l
            h = h_ref[...]
            c = c_ref[...]
            x = x_sc[...]
            p_lin = (jnp.sum(ctx * wp_ctx_ref[...], axis=-1, keepdims=True)
                     + jnp.sum(h * wp_h_ref[...], axis=-1, keepdims=True)
                     + jnp.sum(c * wp_c_ref[...], axis=-1, keepdims=True)
                     + jnp.sum(x * wp_x_ref[...], axis=-1, keepdims=True)
                     + bp_ref[...])
            p_gen = jax.nn.sigmoid(p_lin)
            ctx_ref[...] = ctx
            attn_ref[...] = attn
            attn_oov_ref[...] = (1.0 - p_gen) * attn
            pgen_ref[...] = p_gen
            hidden_ref[...] = mm(h, w1t_h_ref) + mm(ctx, w1t_ctx_ref) + b1_ref[...]

    return _state_kernel


def _make_vocab_kernel(vocab_size, nv, rows_per_core, t_valid, n_q):
    def _vocab_kernel(hidden_ref, pgen_ref, w2t_ref, b2_ref, val_ref, idx_ref,
                      out_ref, logits_sc, m_sc, l_sc, scat_sc):
        f32, bf16, i32 = jnp.float32, jnp.bfloat16, jnp.int32
        p_id = pl.program_id(1)
        TV = w2t_ref.shape[1]
        Tpad = val_ref.shape[1]
        rows_per_phase = rows_per_core // nv
        q_per_slab = n_q // nv

        @pl.when(p_id < nv)
        def _():
            j = p_id
            logits = (jnp.dot(hidden_ref[...].astype(bf16), w2t_ref[...],
                              preferred_element_type=f32) + b2_ref[...])

            @pl.when(j == 0)
            def _():
                m_sc[...] = jnp.full(m_sc.shape, -1e30, f32)
                l_sc[...] = jnp.zeros(l_sc.shape, f32)

            m_old = m_sc[...]
            m_new = jnp.maximum(m_old, jnp.max(logits, axis=-1, keepdims=True))
            l_sc[...] = (l_sc[...] * jnp.exp(m_old - m_new)
                         + jnp.sum(jnp.exp(logits - m_new), axis=-1, keepdims=True))
            m_sc[...] = m_new
            logits_sc[:, pl.ds(pl.multiple_of(j * TV, TV), TV)] = logits

            t_col = jax.lax.broadcasted_iota(i32, (1, Tpad), 1)
            q_iota = jax.lax.broadcasted_iota(i32, (n_q, Tpad), 0)
            r_iota = jax.lax.broadcasted_iota(i32, (128, Tpad), 0)
            for rr in range(rows_per_phase):
                b = j * rows_per_phase + rr
                idx = idx_ref[pl.ds(b, 1), :]
                val = jnp.where(t_col < t_valid, val_ref[pl.ds(b, 1), :], 0.0)
                q = jax.lax.shift_right_logical(idx, 7)
                r = jax.lax.bitwise_and(idx, 127)
                Q = jnp.where(q_iota == q, 1.0, 0.0).astype(bf16)
                Ar = jnp.where(r_iota == r, val, 0.0).astype(bf16)
                R = jax.lax.dot_general(Q, Ar, (((1,), (1,)), ((), ())),
                                        preferred_element_type=f32)
                scat_sc[pl.ds(b, 1)] = R[None]

        @pl.when(p_id >= nv)
        def _():
            f = p_id - nv
            seg = logits_sc[:, pl.ds(pl.multiple_of(f * TV, TV), TV)]
            col = (jax.lax.broadcasted_iota(i32, seg.shape, 1) + f * TV)
            scale = pgen_ref[...] / jnp.maximum(l_sc[...], 1e-30)
            probs = jnp.where(col < vocab_size,
                              jnp.exp(seg - m_sc[...]), 0.0) * scale
            flat = scat_sc[pl.ds(0, scat_sc.shape[0]),
                           pl.ds(f * q_per_slab, q_per_slab), :]
            out_ref[...] = probs + flat.reshape(probs.shape)

    return _vocab_kernel


def kernel(emb, WxT_ctx, WxT_emb, bx, Wih4T, Whh4T, b_lstm4, WsT_h, WsT_c, bs,
           v_att, Wp_ctx, Wp_h, Wp_c, Wp_x, bp, W1T_h, W1T_ctx, b1, W2T, b2,
           enc, encf, mask, decoder_input, h0, c0, previous_context_vector,
           extra_zeros, encoder_input_extra_vocabs):
    f32 = jnp.float32
    V, E = emb.shape
    B, Tpad, twoD = enc.shape
    D = twoD // 2
    T = encoder_input_extra_vocabs.shape[1]
    n_extra = extra_zeros.shape[1]
    Vext = V + n_extra
    n_q = (Vext + 127) // 128
    nv = 4
    TV = (n_q * 128) // nv
    TT = 256
    nT = Tpad // TT
    BH = B // 2

    ctxp = previous_context_vector.astype(f32)
    h0_2 = h0[0].astype(f32)
    c0_2 = c0[0].astype(f32)
    dec_i = decoder_input.astype(jnp.int32)

    def half_spec(a):
        nd = a.ndim
        return pl.BlockSpec((BH,) + a.shape[1:],
                            lambda core, t, _nd=nd: (core,) + (0,) * (_nd - 1))

    def const_spec(a):
        nd = a.ndim
        return pl.BlockSpec(a.shape, lambda core, t, _nd=nd: (0,) * _nd)

    state_inputs = (
        dec_i, emb, ctxp, h0_2, c0_2,
        enc, encf, mask,
        WxT_ctx, WxT_emb, bx,
        Wih4T, Whh4T, b_lstm4,
        WsT_h, WsT_c, bs, v_att,
        Wp_ctx, Wp_h, Wp_c, Wp_x, bp,
        W1T_h, W1T_ctx, b1,
    )
    in_specs = ([pl.BlockSpec(memory_space=pltpu.SMEM),
                 pl.BlockSpec(memory_space=pl.ANY)]
                + [half_spec(a) for a in state_inputs[2:5]]
                + [pl.BlockSpec((BH, TT, twoD), lambda core, t: (core, t, 0)),
                   pl.BlockSpec((BH, TT, twoD), lambda core, t: (core, t, 0)),
                   half_spec(mask)]
                + [const_spec(a) for a in state_inputs[8:]])

    state_out_shapes = (
        jax.ShapeDtypeStruct((B, D), f32),
        jax.ShapeDtypeStruct((B, D), f32),
        jax.ShapeDtypeStruct((B, twoD), f32),
        jax.ShapeDtypeStruct((B, Tpad), f32),
        jax.ShapeDtypeStruct((B, Tpad), f32),
        jax.ShapeDtypeStruct((B, 1), f32),
        jax.ShapeDtypeStruct((B, D), f32),
    )
    out_specs = tuple(half_spec(s) for s in state_out_shapes)

    scratch = [
        pltpu.VMEM((BH, E), f32),
        pltpu.VMEM((BH, E), f32),
        pltpu.VMEM((BH, twoD), f32),
        pltpu.VMEM((BH, 1), f32),
        pltpu.VMEM((BH, 1), f32),
        pltpu.VMEM((BH, twoD), f32),
        pltpu.VMEM((BH, Tpad), f32),
        pltpu.SemaphoreType.DMA,
    ]

    h, c, ctx, attn, attn_oov, p_gen, hidden = pl.pallas_call(
        _make_state_kernel(BH),
        out_shape=state_out_shapes,
        grid=(2, nT),
        in_specs=in_specs,
        out_specs=out_specs,
        scratch_shapes=scratch,
        compiler_params=pltpu.CompilerParams(
            dimension_semantics=("parallel", "arbitrary"),
            vmem_limit_bytes=100 * 1024 * 1024),
    )(*state_inputs)

    idx_pad = jnp.pad(encoder_input_extra_vocabs.astype(jnp.int32),
                      ((0, 0), (0, Tpad - T)))

    vocab_dist_final = pl.pallas_call(
        _make_vocab_kernel(V, nv, BH, T, n_q),
        out_shape=jax.ShapeDtypeStruct((B, Vext), f32),
        grid=(2, 2 * nv),
        in_specs=[pl.BlockSpec((BH, D), lambda core, p: (core, 0)),
                  pl.BlockSpec((BH, 1), lambda core, p: (core, 0)),
                  pl.BlockSpec((D, TV),
                               lambda core, p: (0, jnp.minimum(p, nv - 1))),
                  pl.BlockSpec((1, TV),
                               lambda core, p: (0, jnp.minimum(p, nv - 1))),
                  pl.BlockSpec((BH, Tpad), lambda core, p: (core, 0)),
                  pl.BlockSpec((BH, Tpad), lambda core, p: (core, 0))],
        out_specs=pl.BlockSpec((BH, TV),
                               lambda core, p: (core, jnp.maximum(p - nv, 0))),
        scratch_shapes=[pltpu.VMEM((BH, nv * TV), f32),
                        pltpu.VMEM((BH, 1), f32),
                        pltpu.VMEM((BH, 1), f32),
                        pltpu.VMEM((BH, n_q, 128), f32)],
        compiler_params=pltpu.CompilerParams(
            dimension_semantics=("parallel", "arbitrary"),
            vmem_limit_bytes=100 * 1024 * 1024),
    )(hidden, p_gen, W2T, b2, attn_oov, idx_pad)

    attn_b = attn[:, :T]
    decoder_hidden_next = (h[None], c[None])
    return vocab_dist_final, decoder_hidden_next, ctx, attn_b, None


# speedup vs baseline: 3.0770x; 1.1338x over previous
import jax
import jax.numpy as jnp
from jax.experimental import pallas as pl
from jax.experimental.pallas import tpu as pltpu


def _make_state_kernel(rows_per_core):
    def _state_kernel(dec_ref, emb_ref, ctxprev_ref, h0_ref, c0_ref,
                      enc_ref, encf_ref, mask_ref,
                      wxt_ctx_ref, wxt_emb_ref, bx_ref,
                      wih4_ref, whh4_ref, b4_ref,
                      wst_h_ref, wst_c_ref, bs_ref, v_ref,
                      wp_ctx_ref, wp_h_ref, wp_c_ref, wp_x_ref, bp_ref,
                      w1t_h_ref, w1t_ctx_ref, b1_ref,
                      h_ref, c_ref, ctx_ref, attn_ref, attn_oov_ref,
                      pgen_ref, hidden_ref,
                      emb_sc, x_sc, decf_sc, m_sc, l_sc, ctxacc_sc, scores_sc,
                      emb_sem):
        f32, bf16 = jnp.float32, jnp.bfloat16
        core = pl.program_id(0)
        t = pl.program_id(1)
        n_t = pl.num_programs(1)
        TT = enc_ref.shape[1]

        def mm(a, w_ref):
            return jnp.dot(a.astype(bf16), w_ref[...], preferred_element_type=f32)

        @pl.when(t == 0)
        def _():
            base = core * rows_per_core

            def start_row(b, _):
                row = dec_ref[base + b]
                pltpu.make_async_copy(
                    emb_ref.at[pl.ds(row, 1), :],
                    emb_sc.at[pl.ds(b, 1), :], emb_sem).start()
                return 0

            jax.lax.fori_loop(0, rows_per_core, start_row, 0)

            x_part = mm(ctxprev_ref[...], wxt_ctx_ref) + bx_ref[...]

            def wait_row(b, _):
                pltpu.make_async_copy(
                    emb_ref.at[pl.ds(0, 1), :],
                    emb_sc.at[pl.ds(b, 1), :], emb_sem).wait()
                return 0

            jax.lax.fori_loop(0, rows_per_core, wait_row, 0)

            x = x_part + mm(emb_sc[...], wxt_emb_ref)

            def gate(g):
                return (jnp.dot(x.astype(bf16), wih4_ref[g],
                                preferred_element_type=f32)
                        + jnp.dot(h0_ref[...].astype(bf16), whh4_ref[g],
                                  preferred_element_type=f32)
                        + b4_ref[g])

            i_g = jax.nn.sigmoid(gate(0))
            f_g = jax.nn.sigmoid(gate(1))
            g_g = jnp.tanh(gate(2))
            o_g = jax.nn.sigmoid(gate(3))
            c = f_g * c0_ref[...] + i_g * g_g
            h = o_g * jnp.tanh(c)
            h_ref[...] = h
            c_ref[...] = c
            x_sc[...] = x
            decf_sc[...] = mm(h, wst_h_ref) + mm(c, wst_c_ref) + bs_ref[...]
            m_sc[...] = jnp.full(m_sc.shape, -1e30, f32)
            l_sc[...] = jnp.zeros(l_sc.shape, f32)
            ctxacc_sc[...] = jnp.zeros(ctxacc_sc.shape, f32)

        start = pl.multiple_of(t * TT, TT)
        mask_t = mask_ref[:, pl.ds(start, TT)]
        decf_bf = decf_sc[...].astype(bf16)
        v_bf = v_ref[...].astype(bf16)
        energy = jnp.tanh(encf_ref[...] + decf_bf[:, None, :])
        scores = jnp.sum((energy * v_bf[None, :, :]).astype(f32), axis=-1)
        scores = jnp.where(mask_t > 0.0, scores, -1e30)
        scores_sc[:, pl.ds(start, TT)] = scores

        m_old = m_sc[...]
        m_new = jnp.maximum(m_old, jnp.max(scores, axis=-1, keepdims=True))
        corr = jnp.exp(m_old - m_new)
        p = jnp.exp(scores - m_new)
        l_sc[...] = l_sc[...] * corr + jnp.sum(p, axis=-1, keepdims=True)
        ctxacc_sc[...] = (ctxacc_sc[...] * corr
                          + jnp.sum((p.astype(bf16)[:, :, None]
                                     * enc_ref[...]).astype(f32), axis=1))
        m_sc[...] = m_new

        @pl.when(t == n_t - 1)
        def _():
            inv_l = 1.0 / jnp.maximum(l_sc[...], 1e-9)
            ctx = ctxacc_sc[...] * inv_l
            attn = jnp.exp(scores_sc[...] - m_sc[...]) * inv_l
            h = h_ref[...]
            c = c_ref[...]
            x = x_sc[...]
            p_lin = (jnp.sum(ctx * wp_ctx_ref[...], axis=-1, keepdims=True)
                     + jnp.sum(h * wp_h_ref[...], axis=-1, keepdims=True)
                     + jnp.sum(c * wp_c_ref[...], axis=-1, keepdims=True)
                     + jnp.sum(x * wp_x_ref[...], axis=-1, keepdims=True)
                     + bp_ref[...])
            p_gen = jax.nn.sigmoid(p_lin)
            ctx_ref[...] = ctx
            attn_ref[...] = attn
            attn_oov_ref[...] = (1.0 - p_gen) * attn
            pgen_ref[...] = p_gen
            hidden_ref[...] = mm(h, w1t_h_ref) + mm(ctx, w1t_ctx_ref) + b1_ref[...]

    return _state_kernel


def _make_vocab_kernel(vocab_size, nv, rows_per_core, t_valid, n_q):
    def _vocab_kernel(hidden_ref, pgen_ref, w2t_ref, b2_ref, val_ref, idx_ref,
                      out_ref, logits_sc, m_sc, l_sc, scat_sc):
        f32, bf16, i32 = jnp.float32, jnp.bfloat16, jnp.int32
        p_id = pl.program_id(0)
        TV = w2t_ref.shape[1]
        Tpad = val_ref.shape[1]
        rows_per_phase = rows_per_core // nv
        q_per_slab = n_q // nv

        @pl.when(p_id < nv)
        def _():
            j = p_id
            logits = (jnp.dot(hidden_ref[...].astype(bf16), w2t_ref[...],
                              preferred_element_type=f32) + b2_ref[...])

            @pl.when(j == 0)
            def _():
                m_sc[...] = jnp.full(m_sc.shape, -1e30, f32)
                l_sc[...] = jnp.zeros(l_sc.shape, f32)

            m_old = m_sc[...]
            m_new = jnp.maximum(m_old, jnp.max(logits, axis=-1, keepdims=True))
            l_sc[...] = (l_sc[...] * jnp.exp(m_old - m_new)
                         + jnp.sum(jnp.exp(logits - m_new), axis=-1, keepdims=True))
            m_sc[...] = m_new
            logits_sc[:, pl.ds(pl.multiple_of(j * TV, TV), TV)] = logits

            t_col = jax.lax.broadcasted_iota(i32, (1, Tpad), 1)
            q_iota = jax.lax.broadcasted_iota(i32, (n_q, Tpad), 0)
            r_iota = jax.lax.broadcasted_iota(i32, (128, Tpad), 0)
            for rr in range(rows_per_phase):
                b = j * rows_per_phase + rr
                idx = idx_ref[pl.ds(b, 1), :]
                val = jnp.where(t_col < t_valid, val_ref[pl.ds(b, 1), :], 0.0)
                q = jax.lax.shift_right_logical(idx, 7)
                r = jax.lax.bitwise_and(idx, 127)
                Q = jnp.where(q_iota == q, 1.0, 0.0).astype(bf16)
                Ar = jnp.where(r_iota == r, val, 0.0).astype(bf16)
                R = jax.lax.dot_general(Q, Ar, (((1,), (1,)), ((), ())),
                                        preferred_element_type=f32)
                scat_sc[pl.ds(b, 1)] = R[None]

        @pl.when(p_id >= nv)
        def _():
            f = p_id - nv
            seg = logits_sc[:, pl.ds(pl.multiple_of(f * TV, TV), TV)]
            col = (jax.lax.broadcasted_iota(i32, seg.shape, 1) + f * TV)
            scale = pgen_ref[...] / jnp.maximum(l_sc[...], 1e-30)
            probs = jnp.where(col < vocab_size,
                              jnp.exp(seg - m_sc[...]), 0.0) * scale
            flat = scat_sc[pl.ds(0, scat_sc.shape[0]),
                           pl.ds(f * q_per_slab, q_per_slab), :]
            out_ref[...] = probs + flat.reshape(probs.shape)

    return _vocab_kernel


def kernel(emb, WxT_ctx, WxT_emb, bx, Wih4T, Whh4T, b_lstm4, WsT_h, WsT_c, bs,
           v_att, Wp_ctx, Wp_h, Wp_c, Wp_x, bp, W1T_h, W1T_ctx, b1, W2T, b2,
           enc, encf, mask, decoder_input, h0, c0, previous_context_vector,
           extra_zeros, encoder_input_extra_vocabs):
    f32 = jnp.float32
    V, E = emb.shape
    B, Tpad, twoD = enc.shape
    D = twoD // 2
    T = encoder_input_extra_vocabs.shape[1]
    n_extra = extra_zeros.shape[1]
    Vext = V + n_extra
    n_q = (Vext + 127) // 128
    nv = 4
    TV = (n_q * 128) // nv
    TT = 256
    nT = Tpad // TT
    BH = B // 2

    ctxp = previous_context_vector.astype(f32)
    h0_2 = h0[0].astype(f32)
    c0_2 = c0[0].astype(f32)
    dec_i = decoder_input.astype(jnp.int32)

    def half_spec(a):
        nd = a.ndim
        return pl.BlockSpec((BH,) + a.shape[1:],
                            lambda core, t, _nd=nd: (core,) + (0,) * (_nd - 1))

    def const_spec(a):
        nd = a.ndim
        return pl.BlockSpec(a.shape, lambda core, t, _nd=nd: (0,) * _nd)

    state_inputs = (
        dec_i, emb, ctxp, h0_2, c0_2,
        enc, encf, mask,
        WxT_ctx, WxT_emb, bx,
        Wih4T, Whh4T, b_lstm4,
        WsT_h, WsT_c, bs, v_att,
        Wp_ctx, Wp_h, Wp_c, Wp_x, bp,
        W1T_h, W1T_ctx, b1,
    )
    in_specs = ([pl.BlockSpec(memory_space=pltpu.SMEM),
                 pl.BlockSpec(memory_space=pl.ANY)]
                + [half_spec(a) for a in state_inputs[2:5]]
                + [pl.BlockSpec((BH, TT, twoD), lambda core, t: (core, t, 0)),
                   pl.BlockSpec((BH, TT, twoD), lambda core, t: (core, t, 0)),
                   half_spec(mask)]
                + [const_spec(a) for a in state_inputs[8:]])

    state_out_shapes = (
        jax.ShapeDtypeStruct((B, D), f32),
        jax.ShapeDtypeStruct((B, D), f32),
        jax.ShapeDtypeStruct((B, twoD), f32),
        jax.ShapeDtypeStruct((B, Tpad), f32),
        jax.ShapeDtypeStruct((B, Tpad), f32),
        jax.ShapeDtypeStruct((B, 1), f32),
        jax.ShapeDtypeStruct((B, D), f32),
    )
    out_specs = tuple(half_spec(s) for s in state_out_shapes)

    scratch = [
        pltpu.VMEM((BH, E), f32),
        pltpu.VMEM((BH, E), f32),
        pltpu.VMEM((BH, twoD), f32),
        pltpu.VMEM((BH, 1), f32),
        pltpu.VMEM((BH, 1), f32),
        pltpu.VMEM((BH, twoD), f32),
        pltpu.VMEM((BH, Tpad), f32),
        pltpu.SemaphoreType.DMA,
    ]

    h, c, ctx, attn, attn_oov, p_gen, hidden = pl.pallas_call(
        _make_state_kernel(BH),
        out_shape=state_out_shapes,
        grid=(2, nT),
        in_specs=in_specs,
        out_specs=out_specs,
        scratch_shapes=scratch,
        compiler_params=pltpu.CompilerParams(
            dimension_semantics=("parallel", "arbitrary"),
            vmem_limit_bytes=100 * 1024 * 1024),
    )(*state_inputs)

    idx_pad = jnp.pad(encoder_input_extra_vocabs.astype(jnp.int32),
                      ((0, 0), (0, Tpad - T)))

    vocab_dist_final = pl.pallas_call(
        _make_vocab_kernel(V, nv, B, T, n_q),
        out_shape=jax.ShapeDtypeStruct((B, Vext), f32),
        grid=(2 * nv,),
        in_specs=[pl.BlockSpec((B, D), lambda p: (0, 0)),
                  pl.BlockSpec((B, 1), lambda p: (0, 0)),
                  pl.BlockSpec((D, TV),
                               lambda p: (0, jnp.minimum(p, nv - 1))),
                  pl.BlockSpec((1, TV),
                               lambda p: (0, jnp.minimum(p, nv - 1))),
                  pl.BlockSpec((B, Tpad), lambda p: (0, 0)),
                  pl.BlockSpec((B, Tpad), lambda p: (0, 0))],
        out_specs=pl.BlockSpec((B, TV),
                               lambda p: (0, jnp.maximum(p - nv, 0))),
        scratch_shapes=[pltpu.VMEM((B, nv * TV), f32),
                        pltpu.VMEM((B, 1), f32),
                        pltpu.VMEM((B, 1), f32),
                        pltpu.VMEM((B, n_q, 128), f32)],
        compiler_params=pltpu.CompilerParams(
            dimension_semantics=("arbitrary",),
            vmem_limit_bytes=100 * 1024 * 1024),
    )(hidden, p_gen, W2T, b2, attn_oov, idx_pad)

    attn_b = attn[:, :T]
    decoder_hidden_next = (h[None], c[None])
    return vocab_dist_final, decoder_hidden_next, ctx, attn_b, None
